# baseline (speedup 1.0000x reference)
import jax
import jax.numpy as jnp
from jax.experimental import pallas as pl
from jax.experimental.pallas import tpu as pltpu

N_BINS = 10
BLOCK_R = 256


def _calib_kernel(x_ref, lab_ref, out_ref, acc_ref):
    i = pl.program_id(0)
    nsteps = pl.num_programs(0)

    @pl.when(i == 0)
    def _init():
        acc_ref[...] = jnp.zeros_like(acc_ref)

    x = x_ref[...]
    r, c = x.shape
    m = jnp.max(x, axis=1, keepdims=True)
    s = jnp.sum(jnp.exp(x - m), axis=1, keepdims=True)
    conf = 1.0 / s

    iota_c = jax.lax.broadcasted_iota(jnp.int32, (r, c), 1)
    pred = jnp.min(jnp.where(x == m, iota_c, c), axis=1, keepdims=True)
    labels = lab_ref[...]
    accf = (pred == labels).astype(jnp.float32)

    bounds = (0.10000000149011612, 0.20000000298023224, 0.30000001192092896,
              0.4000000059604645, 0.5, 0.6000000238418579, 0.699999988079071,
              0.800000011920929, 0.9000000357627869)
    b = jnp.zeros((r, 1), dtype=jnp.int32)
    for bv in bounds:
        b = b + (conf > jnp.float32(bv)).astype(jnp.int32)
    onehot = (b == jax.lax.broadcasted_iota(jnp.int32, (r, N_BINS), 1)
              ).astype(jnp.float32)
    cnt = jnp.sum(onehot, axis=0, keepdims=True)
    sconf = jnp.sum(conf * onehot, axis=0, keepdims=True)
    sacc = jnp.sum(accf * onehot, axis=0, keepdims=True)
    acc_ref[...] += jnp.concatenate([cnt, sconf, sacc], axis=0)

    @pl.when(i == nsteps - 1)
    def _finalize():
        stats = acc_ref[...]
        count = stats[0:1, :]
        safe = jnp.maximum(count, 1.0)
        avg_conf = stats[1:2, :] / safe
        avg_acc = stats[2:3, :] / safe
        gap = avg_conf - avg_acc
        n_total = jnp.float32(nsteps) * r
        prop = count / n_total
        nonempty = count > 0.0
        ece = jnp.sum(jnp.where(nonempty, jnp.abs(gap) * prop, 0.0))
        sece = jnp.sum(jnp.where(nonempty, gap * prop, 0.0))
        mce = jnp.max(jnp.where(nonempty, jnp.abs(gap), -jnp.inf))
        lane = jax.lax.broadcasted_iota(jnp.int32, (1, 128), 1)
        vec = jnp.where(lane == 0, ece,
                        jnp.where(lane == 1, sece,
                                  jnp.where(lane == 2, mce, 0.0)))
        out_ref[...] = vec


def kernel(logits, labels):
    n, c = logits.shape
    grid = n // BLOCK_R
    labels2d = labels.reshape(n, 1)
    out = pl.pallas_call(
        _calib_kernel,
        grid=(grid,),
        in_specs=[
            pl.BlockSpec((BLOCK_R, c), lambda i: (i, 0)),
            pl.BlockSpec((BLOCK_R, 1), lambda i: (i, 0)),
        ],
        out_specs=pl.BlockSpec((1, 128), lambda i: (0, 0)),
        out_shape=jax.ShapeDtypeStruct((1, 128), jnp.float32),
        scratch_shapes=[pltpu.VMEM((3, N_BINS), jnp.float32)],
        compiler_params=pltpu.CompilerParams(
            dimension_semantics=("arbitrary",),
        ),
    )(logits, labels2d)
    ece = out[0, 0:1]
    sece = out[0, 1:2]
    mce = out[0, 2]
    return (ece, sece, mce)
